# baseline (speedup 1.0000x reference)
import functools

import jax
import jax.numpy as jnp
from jax import lax
from jax.experimental import pallas as pl
from jax.experimental.pallas import tpu as pltpu
from jax.experimental.pallas import tpu_sc as plsc

C_IN = 256
C_OUT = 128
H = 512
W = 512
HW = H * W
S = 16
CHUNK = HW // S
NROWS = C_OUT * S
NC = 2
NS = 16
NW = NC * NS
R = NROWS // NW
G = 4
NITER = R // G


def _sc_gather(m2, gidx):
    mesh = plsc.VectorSubcoreMesh(core_axis_name="c", subcore_axis_name="s")

    @functools.partial(
        pl.kernel,
        mesh=mesh,
        out_type=jax.ShapeDtypeStruct((NROWS, CHUNK), jnp.float32),
        scratch_types=[
            pltpu.VMEM((NITER, G), jnp.int32),
            pltpu.VMEM((G, CHUNK), jnp.float32),
            pltpu.SemaphoreType.DMA,
        ],
    )
    def k(m_hbm, gidx_hbm, out_hbm, idx_v, buf, sem):
        wid = lax.axis_index("s") * NC + lax.axis_index("c")
        base = wid * R
        pltpu.sync_copy(gidx_hbm.at[wid], idx_v)

        def body(i, carry):
            pltpu.async_copy(m_hbm.at[idx_v.at[i]], buf, sem).wait()
            pltpu.sync_copy(buf, out_hbm.at[pl.ds(base + i * G, G)])
            return carry

        lax.fori_loop(0, NITER, body, 0)

    return k(m2, gidx)


def kernel(model, channel_map):
    m2 = model.reshape(C_IN * S, CHUNK)
    gidx = (
        channel_map[:, None] * S + jnp.arange(S, dtype=jnp.int32)[None, :]
    ).reshape(NW, NITER, G)
    out2 = _sc_gather(m2, gidx)
    return out2.reshape(C_OUT, H, W)


# speedup vs baseline: 1.0136x; 1.0136x over previous
import functools

import jax
import jax.numpy as jnp
from jax import lax
from jax.experimental import pallas as pl
from jax.experimental.pallas import tpu as pltpu
from jax.experimental.pallas import tpu_sc as plsc

C_IN = 256
C_OUT = 128
H = 512
W = 512
HW = H * W
S = 16
CHUNK = HW // S
NROWS = C_OUT * S
NC = 2
NS = 16
NW = NC * NS
R = NROWS // NW
G = 2
NITER = R // G
NBUF = 3


def _sc_gather(m2, gidx):
    mesh = plsc.VectorSubcoreMesh(core_axis_name="c", subcore_axis_name="s")

    @functools.partial(
        pl.kernel,
        mesh=mesh,
        out_type=jax.ShapeDtypeStruct((NROWS, CHUNK), jnp.float32),
        scratch_types=[
            pltpu.VMEM((NITER, G), jnp.int32),
            pltpu.VMEM((NBUF, G, CHUNK), jnp.float32),
            pltpu.SemaphoreType.DMA,
            pltpu.SemaphoreType.DMA,
        ],
    )
    def k(m_hbm, gidx_hbm, out_hbm, idx_v, buf, gsem, wsem):
        wid = lax.axis_index("s") * NC + lax.axis_index("c")
        base = wid * R
        pltpu.sync_copy(gidx_hbm.at[wid], idx_v)

        for b in range(NBUF):
            pltpu.async_copy(m_hbm.at[idx_v.at[b]], buf.at[b], gsem)

        def body(j, carry):
            slot = lax.rem(j, NBUF)
            pltpu.make_async_copy(
                m_hbm.at[idx_v.at[0]], buf.at[0], gsem
            ).wait()
            pltpu.async_copy(
                buf.at[slot], out_hbm.at[pl.ds(base + j * G, G)], wsem
            )
            pltpu.make_async_copy(
                buf.at[0], out_hbm.at[pl.ds(base, G)], wsem
            ).wait()

            @pl.when(j + NBUF < NITER)
            def _():
                pltpu.async_copy(
                    m_hbm.at[idx_v.at[j + NBUF]], buf.at[slot], gsem
                )

            return carry

        lax.fori_loop(0, NITER, body, 0)

    return k(m2, gidx)


def kernel(model, channel_map):
    m2 = model.reshape(C_IN * S, CHUNK)
    gidx = (
        channel_map[:, None] * S + jnp.arange(S, dtype=jnp.int32)[None, :]
    ).reshape(NW, NITER, G)
    out2 = _sc_gather(m2, gidx)
    return out2.reshape(C_OUT, H, W)


# speedup vs baseline: 5.2525x; 5.1823x over previous
import functools

import jax
import jax.numpy as jnp
from jax import lax
from jax.experimental import pallas as pl
from jax.experimental.pallas import tpu as pltpu
from jax.experimental.pallas import tpu_sc as plsc

C_IN = 256
C_OUT = 128
H = 512
W = 512
NC = 2
NS = 16
NW = NC * NS
P = C_OUT // NW
RB = 64
NR = H // RB
NITER = P * NR
NBUF = 3


def _sc_gather(model, cm2):
    mesh = plsc.VectorSubcoreMesh(core_axis_name="c", subcore_axis_name="s")

    @functools.partial(
        pl.kernel,
        mesh=mesh,
        out_type=jax.ShapeDtypeStruct((C_OUT, H, W), jnp.float32),
        scratch_types=[
            pltpu.VMEM((P, 1), jnp.int32),
            pltpu.VMEM((NBUF, 1, RB, W), jnp.float32),
            pltpu.SemaphoreType.DMA,
            pltpu.SemaphoreType.DMA,
        ],
    )
    def k(m_hbm, cm_hbm, out_hbm, idx_v, buf, gsem, wsem):
        wid = lax.axis_index("s") * NC + lax.axis_index("c")
        pbase = wid * P
        pltpu.sync_copy(cm_hbm.at[pl.ds(pbase, P)], idx_v)

        def start_gather(t, slot):
            j = lax.div(t, NR)
            r = lax.rem(t, NR)
            pltpu.async_copy(
                m_hbm.at[idx_v.at[j], pl.ds(r * RB, RB)],
                buf.at[slot],
                gsem,
            )

        for b in range(NBUF):
            start_gather(b, b)

        def body(t, carry):
            slot = lax.rem(t, NBUF)
            j = lax.div(t, NR)
            r = lax.rem(t, NR)
            pltpu.make_async_copy(
                m_hbm.at[idx_v.at[0], pl.ds(0, RB)], buf.at[0], gsem
            ).wait()
            pltpu.async_copy(
                buf.at[slot],
                out_hbm.at[pl.ds(pbase + j, 1), pl.ds(r * RB, RB)],
                wsem,
            )
            pltpu.make_async_copy(
                buf.at[0], out_hbm.at[pl.ds(pbase, 1), pl.ds(0, RB)], wsem
            ).wait()

            @pl.when(t + NBUF < NITER)
            def _():
                start_gather(t + NBUF, slot)

            return carry

        lax.fori_loop(0, NITER, body, 0)

    return k(model, cm2)


def kernel(model, channel_map):
    cm2 = channel_map.reshape(C_OUT, 1)
    return _sc_gather(model, cm2)


# speedup vs baseline: 5.2576x; 1.0010x over previous
import functools

import jax
import jax.numpy as jnp
from jax import lax
from jax.experimental import pallas as pl
from jax.experimental.pallas import tpu as pltpu
from jax.experimental.pallas import tpu_sc as plsc

C_IN = 256
C_OUT = 128
H = 512
W = 512
NC = 2
NS = 16
NW = NC * NS
P = C_OUT // NW
RB = 64
NR = H // RB
NITER = P * NR
NBUF = 3


def _sc_gather(model, cm2):
    mesh = plsc.VectorSubcoreMesh(core_axis_name="c", subcore_axis_name="s")

    @functools.partial(
        pl.kernel,
        mesh=mesh,
        out_type=jax.ShapeDtypeStruct((C_OUT, H, W), jnp.float32),
        scratch_types=[
            pltpu.VMEM((P, 1), jnp.int32),
            pltpu.VMEM((NBUF, 1, RB, W), jnp.float32),
            pltpu.SemaphoreType.DMA,
            pltpu.SemaphoreType.DMA,
        ],
    )
    def k(m_hbm, cm_hbm, out_hbm, idx_v, buf, gsem, wsem):
        wid = lax.axis_index("s") * NC + lax.axis_index("c")
        pbase = wid * P
        pltpu.sync_copy(cm_hbm.at[pl.ds(pbase, P)], idx_v)

        def start_gather(t, slot):
            j = lax.div(t, NR)
            r = lax.rem(t, NR)
            pltpu.async_copy(
                m_hbm.at[idx_v.at[j], pl.ds(r * RB, RB)],
                buf.at[slot],
                gsem,
            )

        for b in range(NBUF - 1):
            start_gather(b, b)

        def body(t, carry):
            slot = lax.rem(t, NBUF)
            j = lax.div(t, NR)
            r = lax.rem(t, NR)
            pltpu.make_async_copy(
                m_hbm.at[idx_v.at[0], pl.ds(0, RB)], buf.at[0], gsem
            ).wait()
            pltpu.async_copy(
                buf.at[slot],
                out_hbm.at[pl.ds(pbase + j, 1), pl.ds(r * RB, RB)],
                wsem,
            )

            @pl.when(t >= 1)
            def _():
                pltpu.make_async_copy(
                    buf.at[0], out_hbm.at[pl.ds(pbase, 1), pl.ds(0, RB)],
                    wsem,
                ).wait()

            @pl.when(t + NBUF - 1 < NITER)
            def _():
                start_gather(t + NBUF - 1, lax.rem(t + NBUF - 1, NBUF))

            return carry

        lax.fori_loop(0, NITER, body, 0)
        pltpu.make_async_copy(
            buf.at[0], out_hbm.at[pl.ds(pbase, 1), pl.ds(0, RB)], wsem
        ).wait()

    return k(model, cm2)


def kernel(model, channel_map):
    cm2 = channel_map.reshape(C_OUT, 1)
    return _sc_gather(model, cm2)


# speedup vs baseline: 5.2619x; 1.0008x over previous
import functools

import jax
import jax.numpy as jnp
from jax import lax
from jax.experimental import pallas as pl
from jax.experimental.pallas import tpu as pltpu
from jax.experimental.pallas import tpu_sc as plsc

C_IN = 256
C_OUT = 128
H = 512
W = 512
NC = 2
NS = 16
NW = NC * NS
P = C_OUT // NW
RB = 32
NR = H // RB
NITER = P * NR
NBUF = 6


def _sc_gather(model, cm2):
    mesh = plsc.VectorSubcoreMesh(core_axis_name="c", subcore_axis_name="s")

    @functools.partial(
        pl.kernel,
        mesh=mesh,
        out_type=jax.ShapeDtypeStruct((C_OUT, H, W), jnp.float32),
        scratch_types=[
            pltpu.VMEM((P, 1), jnp.int32),
            pltpu.VMEM((NBUF, 1, RB, W), jnp.float32),
            pltpu.SemaphoreType.DMA,
            pltpu.SemaphoreType.DMA,
        ],
    )
    def k(m_hbm, cm_hbm, out_hbm, idx_v, buf, gsem, wsem):
        wid = lax.axis_index("s") * NC + lax.axis_index("c")
        pbase = wid * P
        pltpu.sync_copy(cm_hbm.at[pl.ds(pbase, P)], idx_v)

        def start_gather(t, slot):
            j = lax.div(t, NR)
            r = lax.rem(t, NR)
            pltpu.async_copy(
                m_hbm.at[idx_v.at[j], pl.ds(r * RB, RB)],
                buf.at[slot],
                gsem,
            )

        for b in range(NBUF - 1):
            start_gather(b, b)

        def body(t, carry):
            slot = lax.rem(t, NBUF)
            j = lax.div(t, NR)
            r = lax.rem(t, NR)
            pltpu.make_async_copy(
                m_hbm.at[idx_v.at[0], pl.ds(0, RB)], buf.at[0], gsem
            ).wait()
            pltpu.async_copy(
                buf.at[slot],
                out_hbm.at[pl.ds(pbase + j, 1), pl.ds(r * RB, RB)],
                wsem,
            )

            @pl.when(t >= 1)
            def _():
                pltpu.make_async_copy(
                    buf.at[0], out_hbm.at[pl.ds(pbase, 1), pl.ds(0, RB)],
                    wsem,
                ).wait()

            @pl.when(t + NBUF - 1 < NITER)
            def _():
                start_gather(t + NBUF - 1, lax.rem(t + NBUF - 1, NBUF))

            return carry

        lax.fori_loop(0, NITER, body, 0)
        pltpu.make_async_copy(
            buf.at[0], out_hbm.at[pl.ds(pbase, 1), pl.ds(0, RB)], wsem
        ).wait()

    return k(model, cm2)


def kernel(model, channel_map):
    cm2 = channel_map.reshape(C_OUT, 1)
    return _sc_gather(model, cm2)
